# baseline (speedup 1.0000x reference)
import functools

import jax
import jax.numpy as jnp
from jax.experimental import pallas as pl
from jax.experimental.pallas import tpu as pltpu
from jax.experimental.pallas import tpu_sc as plsc

_LANES = 16
_C = 16
_NC = 2
_NS = 16


def _sc_embed(src_flat, token_table, pe, B, S, D):
    N = B * S
    NW = _NC * _NS
    spw = S // NW
    n_chunks = spw // _C
    mesh = plsc.VectorSubcoreMesh(core_axis_name="c", subcore_axis_name="s")

    @functools.partial(
        pl.kernel,
        out_type=jax.ShapeDtypeStruct((N, D), jnp.float32),
        mesh=mesh,
        scratch_types=[
            pltpu.VMEM((B * spw,), jnp.int32),
            pltpu.VMEM((2 * _C, D), jnp.float32),
            pltpu.VMEM((2 * _C, D), jnp.float32),
            pltpu.VMEM((_C, D), jnp.float32),
            pltpu.VMEM((_C, D), jnp.float32),
            pltpu.SemaphoreType.DMA,
            pltpu.SemaphoreType.DMA,
            pltpu.SemaphoreType.DMA,
            pltpu.SemaphoreType.DMA,
            pltpu.SemaphoreType.DMA,
            pltpu.SemaphoreType.DMA,
        ],
    )
    def sc_kernel(table_hbm, idx_hbm, pe_hbm, out_hbm,
                  idx_v, g_a, g_b, pe_a, pe_b,
                  gsem_a, gsem_b, osem_a, osem_b, pesem_a, pesem_b):
        wid = jax.lax.axis_index("s") * _NC + jax.lax.axis_index("c")
        s0 = wid * spw

        def gather_descrs(c, h, buf, sem):
            return [
                pltpu.make_async_copy(
                    table_hbm.at[
                        idx_v.at[pl.ds((2 * h + j) * spw + c * _C, _C)]
                    ],
                    buf.at[pl.ds(j * _C, _C)],
                    sem,
                )
                for j in range(2)
            ]

        def out_descrs(c, h, buf, sem):
            return [
                pltpu.make_async_copy(
                    buf.at[pl.ds(j * _C, _C)],
                    out_hbm.at[pl.ds((2 * h + j) * S + s0 + c * _C, _C)],
                    sem,
                )
                for j in range(2)
            ]

        def pe_descr(c, buf, sem):
            return pltpu.make_async_copy(
                pe_hbm.at[pl.ds(s0 + c * _C, _C)], buf, sem
            )

        def add_half(buf, pe_v, j):
            @pl.loop(0, _C)
            def _row(r):
                for u in range(D // _LANES):
                    sl = pl.ds(u * _LANES, _LANES)
                    plsc.addupdate(buf.at[j * _C + r, sl], pe_v[r, sl])

        idx_descrs = [
            pltpu.make_async_copy(
                idx_hbm.at[pl.ds(b * S + s0, spw)],
                idx_v.at[pl.ds(b * spw, spw)],
                gsem_b,
            )
            for b in range(B)
        ]
        for d in idx_descrs:
            d.start()
        for d in idx_descrs:
            d.wait()

        for d in gather_descrs(0, 0, g_a, gsem_a):
            d.start()
        pe_descr(0, pe_a, pesem_a).start()
        pe_descr(1, pe_b, pesem_b).start()

        def sub_body(c, h, buf, gsem, osem, obuf, ogsem, oosem,
                     pe_v, pesem):
            for d in gather_descrs(c, h, buf, gsem):
                d.wait()
            if h == 0:
                pe_descr(c, pe_v, pesem).wait()
            add_half(buf, pe_v, 0)

            if h == 0:
                @pl.when(c > 0)
                def _():
                    for d in out_descrs(c - 1, 1, obuf, oosem):
                        d.wait()

                for d in gather_descrs(c, 1, obuf, ogsem):
                    d.start()
            else:
                for d in out_descrs(c, 0, obuf, oosem):
                    d.wait()

                @pl.when(c + 1 < n_chunks)
                def _():
                    for d in gather_descrs(c + 1, 0, obuf, ogsem):
                        d.start()

            add_half(buf, pe_v, 1)
            for d in out_descrs(c, h, buf, osem):
                d.start()

            if h == 1:
                @pl.when(c + 2 < n_chunks)
                def _():
                    pe_descr(c + 2, pe_v, pesem).start()

        @pl.loop(0, n_chunks // 2)
        def _pair(p):
            c0 = 2 * p
            c1 = 2 * p + 1
            sub_body(c0, 0, g_a, gsem_a, osem_a, g_b, gsem_b, osem_b,
                     pe_a, pesem_a)
            sub_body(c0, 1, g_b, gsem_b, osem_b, g_a, gsem_a, osem_a,
                     pe_a, pesem_a)
            sub_body(c1, 0, g_a, gsem_a, osem_a, g_b, gsem_b, osem_b,
                     pe_b, pesem_b)
            sub_body(c1, 1, g_b, gsem_b, osem_b, g_a, gsem_a, osem_a,
                     pe_b, pesem_b)

        for d in out_descrs(n_chunks - 1, 1, g_b, osem_b):
            d.wait()

    return sc_kernel(token_table, src_flat, pe)


def kernel(src, token_table, pe):
    B, S = src.shape
    V, D = token_table.shape
    idx = src.reshape(B * S)
    out = _sc_embed(idx, token_table, pe, B, S, D)
    return out.reshape(B, S, D)


# speedup vs baseline: 1.0478x; 1.0478x over previous
import functools

import jax
import jax.numpy as jnp
from jax.experimental import pallas as pl
from jax.experimental.pallas import tpu as pltpu
from jax.experimental.pallas import tpu_sc as plsc

_LANES = 16
_C = 8
_NC = 2
_NS = 16


def _sc_embed(src_flat, token_table, pe, B, S, D):
    N = B * S
    NW = _NC * _NS
    spw = S // NW
    n_chunks = spw // _C
    mesh = plsc.VectorSubcoreMesh(core_axis_name="c", subcore_axis_name="s")

    @functools.partial(
        pl.kernel,
        out_type=jax.ShapeDtypeStruct((N, D), jnp.float32),
        mesh=mesh,
        scratch_types=[
            pltpu.VMEM((B * spw,), jnp.int32),
            pltpu.VMEM((B * _C, D), jnp.float32),
            pltpu.VMEM((B * _C, D), jnp.float32),
            pltpu.VMEM((_C, D), jnp.float32),
            pltpu.VMEM((_C, D), jnp.float32),
            pltpu.SemaphoreType.DMA,
            pltpu.SemaphoreType.DMA,
            pltpu.SemaphoreType.DMA,
            pltpu.SemaphoreType.DMA,
            pltpu.SemaphoreType.DMA,
            pltpu.SemaphoreType.DMA,
        ],
    )
    def sc_kernel(table_hbm, idx_hbm, pe_hbm, out_hbm,
                  idx_v, g_a, g_b, pe_a, pe_b,
                  gsem_a, gsem_b, osem_a, osem_b, pesem_a, pesem_b):
        wid = jax.lax.axis_index("s") * _NC + jax.lax.axis_index("c")
        s0 = wid * spw

        def gather_descr(c, b, buf, sem):
            return pltpu.make_async_copy(
                table_hbm.at[idx_v.at[pl.ds(b * spw + c * _C, _C)]],
                buf.at[pl.ds(b * _C, _C)],
                sem,
            )

        def gather_descrs(c, buf, sem):
            return [gather_descr(c, b, buf, sem) for b in range(B)]

        def out_descrs(c, buf, sem):
            return [
                pltpu.make_async_copy(
                    buf.at[pl.ds(b * _C, _C)],
                    out_hbm.at[pl.ds(b * S + s0 + c * _C, _C)],
                    sem,
                )
                for b in range(B)
            ]

        def pe_descr(c, buf, sem):
            return pltpu.make_async_copy(
                pe_hbm.at[pl.ds(s0 + c * _C, _C)], buf, sem
            )

        def add_batches(buf, pe_v, batches):
            for b in batches:
                @pl.loop(0, _C)
                def _row(r):
                    for u in range(D // _LANES):
                        sl = pl.ds(u * _LANES, _LANES)
                        plsc.addupdate(buf.at[b * _C + r, sl], pe_v[r, sl])

        pe_descr(0, pe_a, pesem_a).start()
        pe_descr(1, pe_b, pesem_b).start()
        idx_descrs = [
            pltpu.make_async_copy(
                idx_hbm.at[pl.ds(b * S + s0, spw)],
                idx_v.at[pl.ds(b * spw, spw)],
                gsem_b,
            )
            for b in range(B)
        ]
        for d in idx_descrs:
            d.start()
        for b in range(B):
            idx_descrs[b].wait()
            gather_descr(0, b, g_a, gsem_a).start()

        def sub_body(c, buf, pe_v, gsem, osem, pesem,
                     obuf, ogsem, oosem):
            for d in gather_descrs(c, buf, gsem):
                d.wait()
            pe_descr(c, pe_v, pesem).wait()
            add_batches(buf, pe_v, (0, 1))

            @pl.when(c > 0)
            def _():
                for d in out_descrs(c - 1, obuf, oosem):
                    d.wait()

            @pl.when(c + 1 < n_chunks)
            def _():
                for d in gather_descrs(c + 1, obuf, ogsem):
                    d.start()

            add_batches(buf, pe_v, (2, 3))
            for d in out_descrs(c, buf, osem):
                d.start()

            @pl.when(c + 2 < n_chunks)
            def _():
                pe_descr(c + 2, pe_v, pesem).start()

        @pl.loop(0, n_chunks // 2)
        def _pair(p):
            c = 2 * p
            sub_body(c, g_a, pe_a, gsem_a, osem_a, pesem_a,
                     g_b, gsem_b, osem_b)
            sub_body(c + 1, g_b, pe_b, gsem_b, osem_b, pesem_b,
                     g_a, gsem_a, osem_a)

        for d in out_descrs(n_chunks - 1, g_b, osem_b):
            d.wait()

    return sc_kernel(token_table, src_flat, pe)


def kernel(src, token_table, pe):
    B, S = src.shape
    V, D = token_table.shape
    idx = src.reshape(B * S)
    out = _sc_embed(idx, token_table, pe, B, S, D)
    return out.reshape(B, S, D)
